# baseline (speedup 1.0000x reference)
import jax
import jax.numpy as jnp
from jax.experimental import pallas as pl

N_NODES = 50000
B = 256
HID = 64
HEADS = 4
DH = HID // HEADS
EPS = 1e-5


def _bn(h, g, b):
    mu = jnp.mean(h, axis=0)
    var = jnp.var(h, axis=0)
    return (h - mu) / jnp.sqrt(var + EPS) * g + b


def _gat(h, src, dst, cp):
    n = h.shape[0]
    xl = (h @ cp['W']).reshape(n, HEADS, DH)
    asrc = jnp.sum(xl * cp['att_src'][None], axis=-1)
    adst = jnp.sum(xl * cp['att_dst'][None], axis=-1)
    alpha = jax.nn.leaky_relu(asrc[src] + adst[dst], 0.2)
    amax = jax.ops.segment_max(alpha, dst, num_segments=n)
    amax = jnp.where(jnp.isfinite(amax), amax, 0.0)
    ex = jnp.exp(alpha - amax[dst])
    den = jax.ops.segment_sum(ex, dst, num_segments=n)
    w = ex / (den[dst] + 1e-16)
    out = jax.ops.segment_sum(xl[src] * w[:, :, None], dst, num_segments=n)
    return out.reshape(n, HID) + cp['bias']


def _mlp_body(g_ref, desc_ref, n_ref,
              dW1, db1, dg1, dbe1, dW2, db2, dg2, dbe2,
              rWi, rbi, rgi, rbei,
              hW0, hb0, hg0, hbe0, hW1, hb1, hg1, hbe1,
              hW2, hb2, hg2, hbe2, rWo, rbo, out_ref):
    def bn_relu(h, g, b):
        mu = jnp.mean(h, axis=0, keepdims=True)
        var = jnp.mean((h - mu) ** 2, axis=0, keepdims=True)
        return jnp.maximum((h - mu) / jnp.sqrt(var + EPS) * g + b, 0.0)

    d = jnp.dot(desc_ref[...], dW1[...], preferred_element_type=jnp.float32)
    d = bn_relu(d + db1[...], dg1[...], dbe1[...])
    d = jnp.dot(d, dW2[...], preferred_element_type=jnp.float32)
    d = bn_relu(d + db2[...], dg2[...], dbe2[...])
    g = g_ref[...]
    r = (jnp.dot(g, rWi[...][:2 * HID], preferred_element_type=jnp.float32)
         + jnp.dot(d, rWi[...][2 * HID:2 * HID + 128], preferred_element_type=jnp.float32)
         + n_ref[...] * rWi[...][2 * HID + 128])
    r = bn_relu(r + rbi[...], rgi[...], rbei[...])
    for W, b, gg, be in ((hW0, hb0, hg0, hbe0), (hW1, hb1, hg1, hbe1),
                         (hW2, hb2, hg2, hbe2)):
        r = jnp.dot(r, W[...], preferred_element_type=jnp.float32)
        r = bn_relu(r + b[...], gg[...], be[...])
    out_ref[...] = jnp.dot(r, rWo[...], preferred_element_type=jnp.float32) + rbo[...]


def _mlp_head(g, descriptors, N, params):
    p = params
    args = [g, descriptors, N[:, None],
            p['dW1'], p['db1'][None], p['dg1'][None], p['dbe1'][None],
            p['dW2'], p['db2'][None], p['dg2'][None], p['dbe2'][None],
            p['rWi'], p['rbi'][None], p['rgi'][None], p['rbei'][None]]
    for hp in p['rhidden']:
        args += [hp['W'], hp['b'][None], hp['g'][None], hp['be'][None]]
    args += [p['rWo'], p['rbo'][None]]
    return pl.pallas_call(
        _mlp_body,
        out_shape=jax.ShapeDtypeStruct((B, 1), jnp.float32),
    )(*args)


def kernel(x, edge_index, edge_attr, batch, N, descriptors, params):
    idx = jax.lax.stop_gradient(x[:, 0]).astype(jnp.int32)
    embedded = params['emb_node'][idx]
    other = x[:, 1:] @ params['W_nf'] + params['b_nf']
    h = jnp.concatenate([embedded, other], axis=1)
    n = h.shape[0]
    loop = jnp.arange(n, dtype=edge_index.dtype)
    src = jnp.concatenate([edge_index[0], loop])
    dst = jnp.concatenate([edge_index[1], loop])
    for cp in params['convs']:
        h = _gat(h, src, dst, cp)
        h = _bn(h, cp['bn_g'], cp['bn_b'])
        h = jax.nn.relu(h)
    ones = jnp.ones((n, 1), jnp.float32)
    cnt = jax.ops.segment_sum(ones, batch, num_segments=B)
    mean = jax.ops.segment_sum(h, batch, num_segments=B) / jnp.maximum(cnt, 1.0)
    mx = jax.ops.segment_max(h, batch, num_segments=B)
    mx = jnp.where(jnp.isfinite(mx), mx, 0.0)
    g = jnp.concatenate([mean, mx], axis=1)
    return _mlp_head(g, descriptors, N, params)


# speedup vs baseline: 39.2204x; 39.2204x over previous
import functools

import jax
import jax.numpy as jnp
from jax import lax
from jax.experimental import pallas as pl
from jax.experimental.pallas import tpu as pltpu
from jax.experimental.pallas import tpu_sc as plsc

N_NODES = 50000
B = 256
HID = 64
HEADS = 4
DH = HID // HEADS
EPS = 1e-5

NPAD = 50176
N_EDGES_TOT = 850000
EBLK = 128
NTILES = 32
CHUNKS = 208
EP = NTILES * CHUNKS * EBLK
ROWS_PER_SUB = NPAD // 16


def _edge_body(src_hbm, dst_hbm, atts_hbm, attd_hbm, xlh_hbm, zeros_hbm,
               acc_hbm, den_hbm,
               srcbuf, dstbuf, rows_s, rows_d, drow, atts_v, attd_v,
               acc_sh, den_sh, sem_s, sem_d):
    c = lax.axis_index("c")
    s = lax.axis_index("s")
    wid = s * 2 + c
    rs = s * ROWS_PER_SUB
    pltpu.sync_copy(zeros_hbm.at[pl.ds(rs, ROWS_PER_SUB)],
                    acc_sh.at[pl.ds(rs, ROWS_PER_SUB)])
    pltpu.sync_copy(zeros_hbm.at[pl.ds(rs, ROWS_PER_SUB)],
                    den_sh.at[pl.ds(rs, ROWS_PER_SUB)])
    pltpu.sync_copy(atts_hbm, atts_v)
    pltpu.sync_copy(attd_hbm, attd_v)
    plsc.subcore_barrier()

    ebase = wid * (CHUNKS * EBLK)

    def chunk(t, carry):
        b = ebase + t * EBLK
        pltpu.sync_copy(src_hbm.at[pl.ds(b, EBLK)], srcbuf)
        pltpu.sync_copy(dst_hbm.at[pl.ds(b, EBLK)], dstbuf)
        cp_s = pltpu.async_copy(xlh_hbm.at[srcbuf], rows_s, sem_s)
        cp_d = pltpu.async_copy(xlh_hbm.at[dstbuf], rows_d, sem_d)
        cp_s.wait()
        cp_d.wait()
        atts = atts_v[...]
        attd = attd_v[...]
        for e in range(EBLK):
            rs_e = rows_s[e, :]
            al = jnp.sum(rs_e * atts + rows_d[e, :] * attd)
            al = jnp.maximum(al, 0.2 * al)
            w = jnp.exp(jnp.broadcast_to(al, (16,)))
            rows_s[e, :] = rs_e * w
            drow[e, :] = w
        pltpu.sync_copy(rows_s, acc_sh.at[dstbuf], add=True)
        pltpu.sync_copy(drow, den_sh.at[dstbuf], add=True)
        return carry

    lax.fori_loop(0, CHUNKS, chunk, 0)
    plsc.subcore_barrier()
    pltpu.sync_copy(acc_sh.at[pl.ds(rs, ROWS_PER_SUB)],
                    acc_hbm.at[c, pl.ds(rs, ROWS_PER_SUB)])
    pltpu.sync_copy(den_sh.at[pl.ds(rs, ROWS_PER_SUB)],
                    den_hbm.at[c, pl.ds(rs, ROWS_PER_SUB)])


def _edge_call(src, dst, att_s, att_d, xlh, zeros):
    mesh = plsc.VectorSubcoreMesh(core_axis_name="c", subcore_axis_name="s")
    f = functools.partial(
        pl.kernel,
        mesh=mesh,
        compiler_params=pltpu.CompilerParams(
            needs_layout_passes=False, use_tc_tiling_on_sc=False),
        out_type=(jax.ShapeDtypeStruct((2, NPAD, DH), jnp.float32),
                  jax.ShapeDtypeStruct((2, NPAD, DH), jnp.float32)),
        scratch_types=[
            pltpu.VMEM((EBLK,), jnp.int32),
            pltpu.VMEM((EBLK,), jnp.int32),
            pltpu.VMEM((EBLK, DH), jnp.float32),
            pltpu.VMEM((EBLK, DH), jnp.float32),
            pltpu.VMEM((EBLK, DH), jnp.float32),
            pltpu.VMEM((DH,), jnp.float32),
            pltpu.VMEM((DH,), jnp.float32),
            pltpu.VMEM_SHARED((NPAD, DH), jnp.float32),
            pltpu.VMEM_SHARED((NPAD, DH), jnp.float32),
            pltpu.SemaphoreType.DMA,
            pltpu.SemaphoreType.DMA,
        ],
    )(_edge_body)
    return f(src, dst, att_s, att_d, xlh, zeros)


def _dense_body(h_ref, W_ref, xl_ref):
    xlb = jnp.dot(h_ref[...], W_ref[...], preferred_element_type=jnp.float32)
    for hd in range(HEADS):
        xl_ref[hd] = xlb[:, hd * DH:(hd + 1) * DH]


def _dense_call(hpad, W):
    blk = 1024
    grid = NPAD // blk
    return pl.pallas_call(
        _dense_body,
        grid=(grid,),
        in_specs=[
            pl.BlockSpec((blk, HID), lambda i: (i, 0)),
            pl.BlockSpec((HID, HID), lambda i: (0, 0)),
        ],
        out_specs=pl.BlockSpec((HEADS, blk, DH), lambda i: (0, i, 0)),
        out_shape=jax.ShapeDtypeStruct((HEADS, NPAD, DH), jnp.float32),
    )(hpad, W)


def _bn(h, g, b):
    mu = jnp.mean(h, axis=0)
    var = jnp.var(h, axis=0)
    return (h - mu) / jnp.sqrt(var + EPS) * g + b


def _mlp_body(g_ref, desc_ref, n_ref,
              dW1, db1, dg1, dbe1, dW2, db2, dg2, dbe2,
              rWi, rbi, rgi, rbei,
              hW0, hb0, hg0, hbe0, hW1, hb1, hg1, hbe1,
              hW2, hb2, hg2, hbe2, rWo, rbo, out_ref):
    def bn_relu(h, g, b):
        mu = jnp.mean(h, axis=0, keepdims=True)
        var = jnp.mean((h - mu) ** 2, axis=0, keepdims=True)
        return jnp.maximum((h - mu) / jnp.sqrt(var + EPS) * g + b, 0.0)

    d = jnp.dot(desc_ref[...], dW1[...], preferred_element_type=jnp.float32)
    d = bn_relu(d + db1[...], dg1[...], dbe1[...])
    d = jnp.dot(d, dW2[...], preferred_element_type=jnp.float32)
    d = bn_relu(d + db2[...], dg2[...], dbe2[...])
    g = g_ref[...]
    r = (jnp.dot(g, rWi[...][:2 * HID], preferred_element_type=jnp.float32)
         + jnp.dot(d, rWi[...][2 * HID:2 * HID + 128], preferred_element_type=jnp.float32)
         + n_ref[...] * rWi[...][2 * HID + 128])
    r = bn_relu(r + rbi[...], rgi[...], rbei[...])
    for W, b, gg, be in ((hW0, hb0, hg0, hbe0), (hW1, hb1, hg1, hbe1),
                         (hW2, hb2, hg2, hbe2)):
        r = jnp.dot(r, W[...], preferred_element_type=jnp.float32)
        r = bn_relu(r + b[...], gg[...], be[...])
    out_ref[...] = jnp.dot(r, rWo[...], preferred_element_type=jnp.float32) + rbo[...]


def _mlp_head(g, descriptors, N, params):
    p = params
    args = [g, descriptors, N[:, None],
            p['dW1'], p['db1'][None], p['dg1'][None], p['dbe1'][None],
            p['dW2'], p['db2'][None], p['dg2'][None], p['dbe2'][None],
            p['rWi'], p['rbi'][None], p['rgi'][None], p['rbei'][None]]
    for hp in p['rhidden']:
        args += [hp['W'], hp['b'][None], hp['g'][None], hp['be'][None]]
    args += [p['rWo'], p['rbo'][None]]
    return pl.pallas_call(
        _mlp_body,
        out_shape=jax.ShapeDtypeStruct((B, 1), jnp.float32),
    )(*args)


def kernel(x, edge_index, edge_attr, batch, N, descriptors, params):
    idx = jax.lax.stop_gradient(x[:, 0]).astype(jnp.int32)
    embedded = params['emb_node'][idx]
    other = x[:, 1:] @ params['W_nf'] + params['b_nf']
    h = jnp.concatenate([embedded, other], axis=1)

    loop = jnp.arange(N_NODES, dtype=jnp.int32)
    pad = jnp.full((EP - N_EDGES_TOT,), N_NODES, jnp.int32)
    src = jnp.concatenate([edge_index[0].astype(jnp.int32), loop, pad])
    dst = jnp.concatenate([edge_index[1].astype(jnp.int32), loop, pad])
    zeros = jnp.zeros((NPAD, DH), jnp.float32)

    for cp in params['convs']:
        hpad = jnp.pad(h, ((0, NPAD - N_NODES), (0, 0)))
        xl = _dense_call(hpad, cp['W'])
        heads = []
        for hd in range(HEADS):
            acc, den = _edge_call(src, dst, cp['att_src'][hd], cp['att_dst'][hd],
                                  xl[hd], zeros)
            num = acc[0, :N_NODES] + acc[1, :N_NODES]
            dsum = den[0, :N_NODES, :1] + den[1, :N_NODES, :1]
            heads.append(num / dsum)
        h = jnp.concatenate(heads, axis=1) + cp['bias']
        h = jax.nn.relu(_bn(h, cp['bn_g'], cp['bn_b']))

    ones = jnp.ones((N_NODES, 1), jnp.float32)
    cnt = jax.ops.segment_sum(ones, batch, num_segments=B)
    mean = jax.ops.segment_sum(h, batch, num_segments=B) / jnp.maximum(cnt, 1.0)
    mx = jax.ops.segment_max(h, batch, num_segments=B)
    mx = jnp.where(jnp.isfinite(mx), mx, 0.0)
    g = jnp.concatenate([mean, mx], axis=1)
    return _mlp_head(g, descriptors, N, params)
